# baseline (speedup 1.0000x reference)
import functools

import jax
import jax.numpy as jnp
from jax import lax
from jax.experimental import pallas as pl
from jax.experimental.pallas import tpu as pltpu
from jax.experimental.pallas import tpu_sc as plsc

N = 10000
E = 320000
NHID = 128
NCLASS = 40
D2P = 128
BN_EPS = 1e-5

NC = 2
NS = 16
NW = NC * NS
L = 16
NP = 10240
STRIPE = NP // NS
K = 128
CH = 80
CHB = 8
NB = CH // CHB
PER_W = CH * K
E_PAD = NW * PER_W

_SC_MESH = dict(
    mesh=plsc.VectorSubcoreMesh(core_axis_name="c", subcore_axis_name="s"),
)


KP = 64
CHP = PER_W // KP
NBP = CHP // CHB


def _make_prop(D):
    def body(hs_hbm, srcp_hbm, dstp_hbm, zero_hbm, out_hbm,
             sA, dA, sB, dB, b0, b1, b2, b3, acc,
             g0, g1, g2, g3, c0, c1, c2, c3):
        bufs = (b0, b1, b2, b3)
        gsem = (g0, g1, g2, g3)
        csem = (c0, c1, c2, c3)
        c = lax.axis_index("c")
        s = lax.axis_index("s")
        w = s * NC + c
        for z in range(STRIPE // K):
            pltpu.sync_copy(zero_hbm, acc.at[pl.ds(s * STRIPE + z * K, K)])
        plsc.subcore_barrier()

        def wait_scat(j, dstv):
            pltpu.make_async_copy(bufs[j], acc.at[dstv.at[0]], c0).wait()

        def do_block(bi, srcv, dstv, first_pred):
            pltpu.sync_copy(srcp_hbm.at[w].at[pl.ds(bi * CHB, CHB)], srcv)
            pltpu.sync_copy(dstp_hbm.at[w].at[pl.ds(bi * CHB, CHB)], dstv)
            for p in range(3):
                pltpu.async_copy(hs_hbm.at[srcv.at[p]], bufs[p], gsem[p])
            for ci in range(CHB):
                j = ci % 4
                pltpu.make_async_copy(hs_hbm.at[srcv.at[ci]], bufs[j],
                                      gsem[j]).wait()
                if first_pred is None or ci > 0:
                    wait_scat(j, dstv)
                else:
                    @pl.when(first_pred)
                    def _():
                        wait_scat(j, dstv)
                pltpu.async_copy(bufs[j], acc.at[dstv.at[ci]], c0, add=True)
                if ci + 3 < CHB:
                    jn = (ci + 3) % 4
                    pltpu.async_copy(hs_hbm.at[srcv.at[ci + 3]], bufs[jn],
                                     gsem[jn])

        def pair_body(pi, carry):
            do_block(pi * 2, sA, dA, pi > 0)
            do_block(pi * 2 + 1, sB, dB, None)
            return carry

        lax.fori_loop(0, NBP // 2, pair_body, 0)
        wait_scat(0, dB)
        plsc.subcore_barrier()
        pltpu.sync_copy(acc.at[pl.ds(s * STRIPE, STRIPE)],
                        out_hbm.at[c].at[pl.ds(s * STRIPE, STRIPE)])

    return pl.kernel(
        body,
        out_type=jax.ShapeDtypeStruct((NC, NP, D), jnp.float32),
        scratch_types=[
            pltpu.VMEM((CHB, KP), jnp.int32),
            pltpu.VMEM((CHB, KP), jnp.int32),
            pltpu.VMEM((CHB, KP), jnp.int32),
            pltpu.VMEM((CHB, KP), jnp.int32),
            pltpu.VMEM((KP, D), jnp.float32),
            pltpu.VMEM((KP, D), jnp.float32),
            pltpu.VMEM((KP, D), jnp.float32),
            pltpu.VMEM((KP, D), jnp.float32),
            pltpu.VMEM_SHARED((NP, D), jnp.float32),
            pltpu.SemaphoreType.DMA,
            pltpu.SemaphoreType.DMA,
            pltpu.SemaphoreType.DMA,
            pltpu.SemaphoreType.DMA,
            pltpu.SemaphoreType.DMA,
            pltpu.SemaphoreType.DMA,
            pltpu.SemaphoreType.DMA,
            pltpu.SemaphoreType.DMA,
        ],
        **_SC_MESH,
    )


_sc_prop128 = _make_prop(NHID)

DEGW = 128


def _deg_body(dstp_hbm, ones_hbm, zero_hbm, out_hbm, dstv, onesv, acc):
    c = lax.axis_index("c")
    s = lax.axis_index("s")
    w = s * NC + c
    pltpu.sync_copy(ones_hbm, onesv)
    for z in range(STRIPE // K):
        pltpu.sync_copy(zero_hbm, acc.at[pl.ds(s * STRIPE + z * K, K)])
    plsc.subcore_barrier()

    def block_body(bi, carry):
        pltpu.sync_copy(dstp_hbm.at[w].at[pl.ds(bi * CHB, CHB)], dstv)

        def chunk(ci, cc2):
            pltpu.sync_copy(onesv, acc.at[dstv.at[ci]], add=True)
            return cc2

        return lax.fori_loop(0, CHB, chunk, carry)

    lax.fori_loop(0, NB, block_body, 0)
    plsc.subcore_barrier()
    pltpu.sync_copy(acc.at[pl.ds(s * STRIPE, STRIPE)],
                    out_hbm.at[c].at[pl.ds(s * STRIPE, STRIPE)])


_sc_deg = pl.kernel(
    _deg_body,
    out_type=jax.ShapeDtypeStruct((NC, NP, DEGW), jnp.float32),
    scratch_types=[
        pltpu.VMEM((CHB, K), jnp.int32),
        pltpu.VMEM((K, DEGW), jnp.float32),
        pltpu.VMEM_SHARED((NP, DEGW), jnp.float32),
    ],
    **_SC_MESH,
)


RB = 1000
GRID = N // RB


def _dinv_body(dp_ref, o_ref):
    deg = dp_ref[0] + dp_ref[1]
    o_ref[...] = lax.rsqrt(deg[:, 0:1] + 1.0)


def _tc_dinv(deg_parts):
    return pl.pallas_call(
        _dinv_body,
        out_shape=jax.ShapeDtypeStruct((NP, 1), jnp.float32),
    )(deg_parts)


def _mm_body(x_ref, w_ref, dv_ref, o_ref):
    h = jnp.dot(x_ref[...], w_ref[...], preferred_element_type=jnp.float32)
    o_ref[...] = h * dv_ref[...]


def _tc_mm_scale(x, W, dinv):
    F, D = W.shape
    return pl.pallas_call(
        _mm_body,
        grid=(GRID,),
        in_specs=[
            pl.BlockSpec((RB, F), lambda i: (i, 0)),
            pl.BlockSpec((F, D), lambda i: (0, 0)),
            pl.BlockSpec((RB, 1), lambda i: (i, 0)),
        ],
        out_specs=pl.BlockSpec((RB, D), lambda i: (i, 0)),
        out_shape=jax.ShapeDtypeStruct((N, D), jnp.float32),
    )(x, W, dinv)


def _comb_body(p0_ref, p1_ref, hs_ref, dv_ref, b_ref, t_ref, st_ref):
    i = pl.program_id(0)
    t = (p0_ref[0] + p1_ref[0] + hs_ref[...]) * dv_ref[...] + b_ref[...]
    t_ref[...] = t
    s1 = jnp.sum(t, axis=0, keepdims=True)
    s2 = jnp.sum(t * t, axis=0, keepdims=True)
    st = jnp.concatenate([s1, s2], axis=0)

    @pl.when(i == 0)
    def _():
        st_ref[...] = st

    @pl.when(i > 0)
    def _():
        st_ref[...] += st


def _tc_combine(p, hs, dinv, b):
    D = hs.shape[1]
    return pl.pallas_call(
        _comb_body,
        grid=(GRID,),
        in_specs=[
            pl.BlockSpec((1, RB, D), lambda i: (0, i, 0)),
            pl.BlockSpec((1, RB, D), lambda i: (1, i, 0)),
            pl.BlockSpec((RB, D), lambda i: (i, 0)),
            pl.BlockSpec((RB, 1), lambda i: (i, 0)),
            pl.BlockSpec((1, D), lambda i: (0, 0)),
        ],
        out_specs=[
            pl.BlockSpec((RB, D), lambda i: (i, 0)),
            pl.BlockSpec((2, D), lambda i: (0, 0)),
        ],
        out_shape=[
            jax.ShapeDtypeStruct((N, D), jnp.float32),
            jax.ShapeDtypeStruct((2, D), jnp.float32),
        ],
    )(p, p, hs, dinv, b)


def _comb_final_body(p0_ref, p1_ref, hs_ref, dv_ref, b_ref, t_ref):
    t_ref[...] = (p0_ref[0] + p1_ref[0] + hs_ref[...]) * dv_ref[...] + b_ref[...]


def _tc_combine_final(p, hs, dinv, b):
    D = hs.shape[1]
    return pl.pallas_call(
        _comb_final_body,
        grid=(GRID,),
        in_specs=[
            pl.BlockSpec((1, RB, D), lambda i: (0, i, 0)),
            pl.BlockSpec((1, RB, D), lambda i: (1, i, 0)),
            pl.BlockSpec((RB, D), lambda i: (i, 0)),
            pl.BlockSpec((RB, 1), lambda i: (i, 0)),
            pl.BlockSpec((1, D), lambda i: (0, 0)),
        ],
        out_specs=pl.BlockSpec((RB, D), lambda i: (i, 0)),
        out_shape=jax.ShapeDtypeStruct((N, D), jnp.float32),
    )(p, p, hs, dinv, b)


def _bn_mm_body(st_ref, t_ref, g_ref, be_ref, w_ref, dv_ref, o_ref):
    s1 = st_ref[0:1, :]
    s2 = st_ref[1:2, :]
    mean = s1 * (1.0 / N)
    var = s2 * (1.0 / N) - mean * mean
    inv = lax.rsqrt(var + BN_EPS)
    a = g_ref[...] * inv
    cc = be_ref[...] - mean * a
    h = jnp.maximum(t_ref[...] * a + cc, 0.0)
    o_ref[...] = jnp.dot(h, w_ref[...], preferred_element_type=jnp.float32) * dv_ref[...]


def _tc_bn_mm(st, t, g, be, W, dinv):
    F, D = W.shape
    return pl.pallas_call(
        _bn_mm_body,
        grid=(GRID,),
        in_specs=[
            pl.BlockSpec((2, F), lambda i: (0, 0)),
            pl.BlockSpec((RB, F), lambda i: (i, 0)),
            pl.BlockSpec((1, F), lambda i: (0, 0)),
            pl.BlockSpec((1, F), lambda i: (0, 0)),
            pl.BlockSpec((F, D), lambda i: (0, 0)),
            pl.BlockSpec((RB, 1), lambda i: (i, 0)),
        ],
        out_specs=pl.BlockSpec((RB, D), lambda i: (i, 0)),
        out_shape=jax.ShapeDtypeStruct((N, D), jnp.float32),
    )(st, t, g, be, W, dinv)


def kernel(x, edge_idx, W0, b0, g0, be0, W1, b1, g1, be1, W2, b2):
    src = edge_idx[0]
    dst = edge_idx[1]
    pad = E_PAD - E
    srcf = jnp.concatenate([src, jnp.zeros((pad,), jnp.int32)])
    dstf = jnp.concatenate([dst, jnp.full((pad,), NP - 1, jnp.int32)])
    srcp = srcf.reshape(NW, CHP, KP)
    dstp = dstf.reshape(NW, CHP, KP)
    dstp128 = dstf.reshape(NW, CH, K)
    zero128 = jnp.zeros((K, NHID), jnp.float32)
    ones_blk = jnp.ones((K, DEGW), jnp.float32)
    zero_blk = jnp.zeros((K, DEGW), jnp.float32)

    deg_parts = _sc_deg(dstp128, ones_blk, zero_blk)
    dinv = _tc_dinv(deg_parts)

    hs0 = _tc_mm_scale(x, W0, dinv)
    p0 = _sc_prop128(hs0, srcp, dstp, zero128)
    t0, st0 = _tc_combine(p0, hs0, dinv, b0.reshape(1, -1))

    hs1 = _tc_bn_mm(st0, t0, g0.reshape(1, -1), be0.reshape(1, -1), W1, dinv)
    p1 = _sc_prop128(hs1, srcp, dstp, zero128)
    t1, st1 = _tc_combine(p1, hs1, dinv, b1.reshape(1, -1))

    W2p = jnp.pad(W2, ((0, 0), (0, D2P - NCLASS)))
    b2p = jnp.pad(b2, (0, D2P - NCLASS)).reshape(1, -1)
    hs2 = _tc_bn_mm(st1, t1, g1.reshape(1, -1), be1.reshape(1, -1), W2p, dinv)
    p2 = _sc_prop128(hs2, srcp, dstp, zero128)
    out = _tc_combine_final(p2, hs2, dinv, b2p)

    return out[:, :NCLASS]


# speedup vs baseline: 1.0963x; 1.0963x over previous
import functools

import jax
import jax.numpy as jnp
from jax import lax
from jax.experimental import pallas as pl
from jax.experimental.pallas import tpu as pltpu
from jax.experimental.pallas import tpu_sc as plsc

N = 10000
E = 320000
NHID = 128
NCLASS = 40
D2P = 128
BN_EPS = 1e-5

NC = 2
NS = 16
NW = NC * NS
L = 16
NP = 10240
STRIPE = NP // NS
K = 128
CH = 80
CHB = 8
NB = CH // CHB
PER_W = CH * K
E_PAD = NW * PER_W

_SC_MESH = dict(
    mesh=plsc.VectorSubcoreMesh(core_axis_name="c", subcore_axis_name="s"),
)


def _make_prop(D):
    def body(hs_hbm, srcp_hbm, dstp_hbm, zero_hbm, out_hbm,
             sA, dA, sB, dB, b0, b1, acc, g0, g1, c0):
        bufs = (b0, b1)
        gsem = (g0, g1)
        c = lax.axis_index("c")
        s = lax.axis_index("s")
        w = s * NC + c
        for z in range(STRIPE // K):
            pltpu.sync_copy(zero_hbm, acc.at[pl.ds(s * STRIPE + z * K, K)])
        plsc.subcore_barrier()

        def wait_scat(dstv):
            pltpu.make_async_copy(b0, acc.at[dstv.at[0]], c0).wait()

        def do_block(bi, srcv, dstv, first_pred):
            pltpu.sync_copy(srcp_hbm.at[w].at[pl.ds(bi * CHB, CHB)], srcv)
            pltpu.sync_copy(dstp_hbm.at[w].at[pl.ds(bi * CHB, CHB)], dstv)
            pltpu.async_copy(hs_hbm.at[srcv.at[0]], b0, g0)
            for ci in range(CHB):
                j = ci % 2
                pltpu.make_async_copy(hs_hbm.at[srcv.at[ci]], bufs[j],
                                      gsem[j]).wait()
                if first_pred is None or ci > 0:
                    wait_scat(dstv)
                else:
                    @pl.when(first_pred)
                    def _():
                        wait_scat(dstv)
                pltpu.async_copy(bufs[j], acc.at[dstv.at[ci]], c0, add=True)
                if ci + 1 < CHB:
                    pltpu.async_copy(hs_hbm.at[srcv.at[ci + 1]],
                                     bufs[1 - j], gsem[1 - j])

        def pair_body(pi, carry):
            do_block(pi * 2, sA, dA, pi > 0)
            do_block(pi * 2 + 1, sB, dB, None)
            return carry

        lax.fori_loop(0, NB // 2, pair_body, 0)
        wait_scat(dB)
        plsc.subcore_barrier()
        pltpu.sync_copy(acc.at[pl.ds(s * STRIPE, STRIPE)],
                        out_hbm.at[c].at[pl.ds(s * STRIPE, STRIPE)])

    return pl.kernel(
        body,
        out_type=jax.ShapeDtypeStruct((NC, NP, D), jnp.float32),
        scratch_types=[
            pltpu.VMEM((CHB, K), jnp.int32),
            pltpu.VMEM((CHB, K), jnp.int32),
            pltpu.VMEM((CHB, K), jnp.int32),
            pltpu.VMEM((CHB, K), jnp.int32),
            pltpu.VMEM((K, D), jnp.float32),
            pltpu.VMEM((K, D), jnp.float32),
            pltpu.VMEM_SHARED((NP, D), jnp.float32),
            pltpu.SemaphoreType.DMA,
            pltpu.SemaphoreType.DMA,
            pltpu.SemaphoreType.DMA,
        ],
        **_SC_MESH,
    )


_sc_prop128 = _make_prop(NHID)

DEGW = 128


def _deg_body(dstp_hbm, ones_hbm, zero_hbm, out_hbm, dstv, onesv, acc):
    c = lax.axis_index("c")
    s = lax.axis_index("s")
    w = s * NC + c
    pltpu.sync_copy(ones_hbm, onesv)
    for z in range(STRIPE // K):
        pltpu.sync_copy(zero_hbm, acc.at[pl.ds(s * STRIPE + z * K, K)])
    plsc.subcore_barrier()

    def block_body(bi, carry):
        pltpu.sync_copy(dstp_hbm.at[w].at[pl.ds(bi * CHB, CHB)], dstv)

        def chunk(ci, cc2):
            pltpu.sync_copy(onesv, acc.at[dstv.at[ci]], add=True)
            return cc2

        return lax.fori_loop(0, CHB, chunk, carry)

    lax.fori_loop(0, NB, block_body, 0)
    plsc.subcore_barrier()
    pltpu.sync_copy(acc.at[pl.ds(s * STRIPE, STRIPE)],
                    out_hbm.at[c].at[pl.ds(s * STRIPE, STRIPE)])


_sc_deg = pl.kernel(
    _deg_body,
    out_type=jax.ShapeDtypeStruct((NC, NP, DEGW), jnp.float32),
    scratch_types=[
        pltpu.VMEM((CHB, K), jnp.int32),
        pltpu.VMEM((K, DEGW), jnp.float32),
        pltpu.VMEM_SHARED((NP, DEGW), jnp.float32),
    ],
    **_SC_MESH,
)


RB = 1000
GRID = N // RB


def _dinv_body(dp_ref, o_ref):
    deg = dp_ref[0] + dp_ref[1]
    o_ref[...] = lax.rsqrt(deg[:, 0:1] + 1.0)


def _tc_dinv(deg_parts):
    return pl.pallas_call(
        _dinv_body,
        out_shape=jax.ShapeDtypeStruct((NP, 1), jnp.float32),
    )(deg_parts)


def _mm_body(x_ref, w_ref, dv_ref, o_ref):
    h = jnp.dot(x_ref[...], w_ref[...], preferred_element_type=jnp.float32)
    o_ref[...] = h * dv_ref[...]


def _tc_mm_scale(x, W, dinv):
    F, D = W.shape
    return pl.pallas_call(
        _mm_body,
        grid=(GRID,),
        in_specs=[
            pl.BlockSpec((RB, F), lambda i: (i, 0)),
            pl.BlockSpec((F, D), lambda i: (0, 0)),
            pl.BlockSpec((RB, 1), lambda i: (i, 0)),
        ],
        out_specs=pl.BlockSpec((RB, D), lambda i: (i, 0)),
        out_shape=jax.ShapeDtypeStruct((N, D), jnp.float32),
    )(x, W, dinv)


def _comb_body(p0_ref, p1_ref, hs_ref, dv_ref, b_ref, t_ref, st_ref):
    i = pl.program_id(0)
    t = (p0_ref[0] + p1_ref[0] + hs_ref[...]) * dv_ref[...] + b_ref[...]
    t_ref[...] = t
    s1 = jnp.sum(t, axis=0, keepdims=True)
    s2 = jnp.sum(t * t, axis=0, keepdims=True)
    st = jnp.concatenate([s1, s2], axis=0)

    @pl.when(i == 0)
    def _():
        st_ref[...] = st

    @pl.when(i > 0)
    def _():
        st_ref[...] += st


def _tc_combine(p, hs, dinv, b):
    D = hs.shape[1]
    return pl.pallas_call(
        _comb_body,
        grid=(GRID,),
        in_specs=[
            pl.BlockSpec((1, RB, D), lambda i: (0, i, 0)),
            pl.BlockSpec((1, RB, D), lambda i: (1, i, 0)),
            pl.BlockSpec((RB, D), lambda i: (i, 0)),
            pl.BlockSpec((RB, 1), lambda i: (i, 0)),
            pl.BlockSpec((1, D), lambda i: (0, 0)),
        ],
        out_specs=[
            pl.BlockSpec((RB, D), lambda i: (i, 0)),
            pl.BlockSpec((2, D), lambda i: (0, 0)),
        ],
        out_shape=[
            jax.ShapeDtypeStruct((N, D), jnp.float32),
            jax.ShapeDtypeStruct((2, D), jnp.float32),
        ],
    )(p, p, hs, dinv, b)


def _comb_final_body(p0_ref, p1_ref, hs_ref, dv_ref, b_ref, t_ref):
    t_ref[...] = (p0_ref[0] + p1_ref[0] + hs_ref[...]) * dv_ref[...] + b_ref[...]


def _tc_combine_final(p, hs, dinv, b):
    D = hs.shape[1]
    return pl.pallas_call(
        _comb_final_body,
        grid=(GRID,),
        in_specs=[
            pl.BlockSpec((1, RB, D), lambda i: (0, i, 0)),
            pl.BlockSpec((1, RB, D), lambda i: (1, i, 0)),
            pl.BlockSpec((RB, D), lambda i: (i, 0)),
            pl.BlockSpec((RB, 1), lambda i: (i, 0)),
            pl.BlockSpec((1, D), lambda i: (0, 0)),
        ],
        out_specs=pl.BlockSpec((RB, D), lambda i: (i, 0)),
        out_shape=jax.ShapeDtypeStruct((N, D), jnp.float32),
    )(p, p, hs, dinv, b)


def _bn_mm_body(st_ref, t_ref, g_ref, be_ref, w_ref, dv_ref, o_ref):
    s1 = st_ref[0:1, :]
    s2 = st_ref[1:2, :]
    mean = s1 * (1.0 / N)
    var = s2 * (1.0 / N) - mean * mean
    inv = lax.rsqrt(var + BN_EPS)
    a = g_ref[...] * inv
    cc = be_ref[...] - mean * a
    h = jnp.maximum(t_ref[...] * a + cc, 0.0)
    o_ref[...] = jnp.dot(h, w_ref[...], preferred_element_type=jnp.float32) * dv_ref[...]


def _tc_bn_mm(st, t, g, be, W, dinv):
    F, D = W.shape
    return pl.pallas_call(
        _bn_mm_body,
        grid=(GRID,),
        in_specs=[
            pl.BlockSpec((2, F), lambda i: (0, 0)),
            pl.BlockSpec((RB, F), lambda i: (i, 0)),
            pl.BlockSpec((1, F), lambda i: (0, 0)),
            pl.BlockSpec((1, F), lambda i: (0, 0)),
            pl.BlockSpec((F, D), lambda i: (0, 0)),
            pl.BlockSpec((RB, 1), lambda i: (i, 0)),
        ],
        out_specs=pl.BlockSpec((RB, D), lambda i: (i, 0)),
        out_shape=jax.ShapeDtypeStruct((N, D), jnp.float32),
    )(st, t, g, be, W, dinv)


def kernel(x, edge_idx, W0, b0, g0, be0, W1, b1, g1, be1, W2, b2):
    src = edge_idx[0]
    dst = edge_idx[1]
    pad = E_PAD - E
    srcp = jnp.concatenate([src, jnp.zeros((pad,), jnp.int32)]).reshape(NW, CH, K)
    dstp = jnp.concatenate([dst, jnp.full((pad,), NP - 1, jnp.int32)]).reshape(NW, CH, K)
    zero128 = jnp.zeros((K, NHID), jnp.float32)
    ones_blk = jnp.ones((K, DEGW), jnp.float32)
    zero_blk = jnp.zeros((K, DEGW), jnp.float32)

    deg_parts = _sc_deg(dstp, ones_blk, zero_blk)
    dinv = _tc_dinv(deg_parts)

    hs0 = _tc_mm_scale(x, W0, dinv)
    p0 = _sc_prop128(hs0, srcp, dstp, zero128)
    t0, st0 = _tc_combine(p0, hs0, dinv, b0.reshape(1, -1))

    hs1 = _tc_bn_mm(st0, t0, g0.reshape(1, -1), be0.reshape(1, -1), W1, dinv)
    p1 = _sc_prop128(hs1, srcp, dstp, zero128)
    t1, st1 = _tc_combine(p1, hs1, dinv, b1.reshape(1, -1))

    W2p = jnp.pad(W2, ((0, 0), (0, D2P - NCLASS)))
    b2p = jnp.pad(b2, (0, D2P - NCLASS)).reshape(1, -1)
    hs2 = _tc_bn_mm(st1, t1, g1.reshape(1, -1), be1.reshape(1, -1), W2p, dinv)
    p2 = _sc_prop128(hs2, srcp, dstp, zero128)
    out = _tc_combine_final(p2, hs2, dinv, b2p)

    return out[:, :NCLASS]


# speedup vs baseline: 1.1555x; 1.0540x over previous
import functools

import jax
import jax.numpy as jnp
from jax import lax
from jax.experimental import pallas as pl
from jax.experimental.pallas import tpu as pltpu
from jax.experimental.pallas import tpu_sc as plsc

N = 10000
E = 320000
NHID = 128
NCLASS = 40
D2P = 128
BN_EPS = 1e-5

NC = 2
NS = 16
NW = NC * NS
L = 16
NP = 10240
STRIPE = NP // NS
K = 128
CH = 80
CHB = 8
NB = CH // CHB
PER_W = CH * K
E_PAD = NW * PER_W

_SC_MESH = dict(
    mesh=plsc.VectorSubcoreMesh(core_axis_name="c", subcore_axis_name="s"),
)


def _make_prop(D):
    def body(hs_hbm, srcp_hbm, dstp_hbm, zero_hbm, out_hbm,
             sA, dA, sB, dB, b0, b1, acc, g0, g1, iA, iB):
        c = lax.axis_index("c")
        s = lax.axis_index("s")
        w = s * NC + c
        for z in range(STRIPE // K):
            pltpu.sync_copy(zero_hbm, acc.at[pl.ds(s * STRIPE + z * K, K)])
        plsc.subcore_barrier()

        def idx_issue(bi, sv, dv, isem):
            pltpu.async_copy(srcp_hbm.at[w].at[pl.ds(bi * CHB, CHB)], sv, isem)
            pltpu.async_copy(dstp_hbm.at[w].at[pl.ds(bi * CHB, CHB)], dv, isem)

        def idx_wait(bi, sv, dv, isem):
            pltpu.make_async_copy(srcp_hbm.at[w].at[pl.ds(bi * CHB, CHB)],
                                  sv, isem).wait()
            pltpu.make_async_copy(dstp_hbm.at[w].at[pl.ds(bi * CHB, CHB)],
                                  dv, isem).wait()

        pltpu.sync_copy(srcp_hbm.at[w].at[pl.ds(0, CHB)], sA)
        pltpu.sync_copy(dstp_hbm.at[w].at[pl.ds(0, CHB)], dA)
        pltpu.async_copy(hs_hbm.at[sA.at[0]], b0, g0)
        pltpu.async_copy(hs_hbm.at[sA.at[1]], b1, g1)

        def do_block(bi, srcv, dstv, nsv, ndv, nisem):
            @pl.when(bi + 1 < NB)
            def _():
                idx_issue(bi + 1, nsv, ndv, nisem)

            for ci in range(CHB):
                buf, sem = (b0, g0) if ci % 2 == 0 else (b1, g1)
                pltpu.make_async_copy(hs_hbm.at[srcv.at[ci]], buf, sem).wait()
                pltpu.sync_copy(buf, acc.at[dstv.at[ci]], add=True)
                if ci + 2 < CHB:
                    pltpu.async_copy(hs_hbm.at[srcv.at[ci + 2]], buf, sem)
                elif ci == CHB - 2:
                    @pl.when(bi + 1 < NB)
                    def _(buf=buf, sem=sem):
                        idx_wait(bi + 1, nsv, ndv, nisem)
                        pltpu.async_copy(hs_hbm.at[nsv.at[0]], buf, sem)
                else:
                    @pl.when(bi + 1 < NB)
                    def _(buf=buf, sem=sem):
                        pltpu.async_copy(hs_hbm.at[nsv.at[1]], buf, sem)

        def pair_body(pi, carry):
            do_block(pi * 2, sA, dA, sB, dB, iB)
            do_block(pi * 2 + 1, sB, dB, sA, dA, iA)
            return carry

        lax.fori_loop(0, NB // 2, pair_body, 0)
        plsc.subcore_barrier()
        pltpu.sync_copy(acc.at[pl.ds(s * STRIPE, STRIPE)],
                        out_hbm.at[c].at[pl.ds(s * STRIPE, STRIPE)])

    return pl.kernel(
        body,
        out_type=jax.ShapeDtypeStruct((NC, NP, D), jnp.float32),
        scratch_types=[
            pltpu.VMEM((CHB, K), jnp.int32),
            pltpu.VMEM((CHB, K), jnp.int32),
            pltpu.VMEM((CHB, K), jnp.int32),
            pltpu.VMEM((CHB, K), jnp.int32),
            pltpu.VMEM((K, D), jnp.float32),
            pltpu.VMEM((K, D), jnp.float32),
            pltpu.VMEM_SHARED((NP, D), jnp.float32),
            pltpu.SemaphoreType.DMA,
            pltpu.SemaphoreType.DMA,
            pltpu.SemaphoreType.DMA,
            pltpu.SemaphoreType.DMA,
        ],
        **_SC_MESH,
    )


_sc_prop128 = _make_prop(NHID)

DEGW = 128


def _deg_body(dstp_hbm, ones_hbm, zero_hbm, out_hbm, dstv, onesv, acc):
    c = lax.axis_index("c")
    s = lax.axis_index("s")
    w = s * NC + c
    pltpu.sync_copy(ones_hbm, onesv)
    for z in range(STRIPE // K):
        pltpu.sync_copy(zero_hbm, acc.at[pl.ds(s * STRIPE + z * K, K)])
    plsc.subcore_barrier()

    def block_body(bi, carry):
        pltpu.sync_copy(dstp_hbm.at[w].at[pl.ds(bi * CHB, CHB)], dstv)

        def chunk(ci, cc2):
            pltpu.sync_copy(onesv, acc.at[dstv.at[ci]], add=True)
            return cc2

        return lax.fori_loop(0, CHB, chunk, carry)

    lax.fori_loop(0, NB, block_body, 0)
    plsc.subcore_barrier()
    pltpu.sync_copy(acc.at[pl.ds(s * STRIPE, STRIPE)],
                    out_hbm.at[c].at[pl.ds(s * STRIPE, STRIPE)])


_sc_deg = pl.kernel(
    _deg_body,
    out_type=jax.ShapeDtypeStruct((NC, NP, DEGW), jnp.float32),
    scratch_types=[
        pltpu.VMEM((CHB, K), jnp.int32),
        pltpu.VMEM((K, DEGW), jnp.float32),
        pltpu.VMEM_SHARED((NP, DEGW), jnp.float32),
    ],
    **_SC_MESH,
)


RB = 1000
GRID = N // RB


def _dinv_body(dp_ref, o_ref):
    deg = dp_ref[0] + dp_ref[1]
    o_ref[...] = lax.rsqrt(deg[:, 0:1] + 1.0)


def _tc_dinv(deg_parts):
    return pl.pallas_call(
        _dinv_body,
        out_shape=jax.ShapeDtypeStruct((NP, 1), jnp.float32),
    )(deg_parts)


def _mm_body(x_ref, w_ref, dv_ref, o_ref):
    h = jnp.dot(x_ref[...], w_ref[...], preferred_element_type=jnp.float32)
    o_ref[...] = h * dv_ref[...]


def _tc_mm_scale(x, W, dinv):
    F, D = W.shape
    return pl.pallas_call(
        _mm_body,
        grid=(GRID,),
        in_specs=[
            pl.BlockSpec((RB, F), lambda i: (i, 0)),
            pl.BlockSpec((F, D), lambda i: (0, 0)),
            pl.BlockSpec((RB, 1), lambda i: (i, 0)),
        ],
        out_specs=pl.BlockSpec((RB, D), lambda i: (i, 0)),
        out_shape=jax.ShapeDtypeStruct((N, D), jnp.float32),
    )(x, W, dinv)


def _comb_body(p0_ref, p1_ref, hs_ref, dv_ref, b_ref, t_ref, st_ref):
    i = pl.program_id(0)
    t = (p0_ref[0] + p1_ref[0] + hs_ref[...]) * dv_ref[...] + b_ref[...]
    t_ref[...] = t
    s1 = jnp.sum(t, axis=0, keepdims=True)
    s2 = jnp.sum(t * t, axis=0, keepdims=True)
    st = jnp.concatenate([s1, s2], axis=0)

    @pl.when(i == 0)
    def _():
        st_ref[...] = st

    @pl.when(i > 0)
    def _():
        st_ref[...] += st


def _tc_combine(p, hs, dinv, b):
    D = hs.shape[1]
    return pl.pallas_call(
        _comb_body,
        grid=(GRID,),
        in_specs=[
            pl.BlockSpec((1, RB, D), lambda i: (0, i, 0)),
            pl.BlockSpec((1, RB, D), lambda i: (1, i, 0)),
            pl.BlockSpec((RB, D), lambda i: (i, 0)),
            pl.BlockSpec((RB, 1), lambda i: (i, 0)),
            pl.BlockSpec((1, D), lambda i: (0, 0)),
        ],
        out_specs=[
            pl.BlockSpec((RB, D), lambda i: (i, 0)),
            pl.BlockSpec((2, D), lambda i: (0, 0)),
        ],
        out_shape=[
            jax.ShapeDtypeStruct((N, D), jnp.float32),
            jax.ShapeDtypeStruct((2, D), jnp.float32),
        ],
    )(p, p, hs, dinv, b)


def _comb_final_body(p0_ref, p1_ref, hs_ref, dv_ref, b_ref, t_ref):
    t_ref[...] = (p0_ref[0] + p1_ref[0] + hs_ref[...]) * dv_ref[...] + b_ref[...]


def _tc_combine_final(p, hs, dinv, b):
    D = hs.shape[1]
    return pl.pallas_call(
        _comb_final_body,
        grid=(GRID,),
        in_specs=[
            pl.BlockSpec((1, RB, D), lambda i: (0, i, 0)),
            pl.BlockSpec((1, RB, D), lambda i: (1, i, 0)),
            pl.BlockSpec((RB, D), lambda i: (i, 0)),
            pl.BlockSpec((RB, 1), lambda i: (i, 0)),
            pl.BlockSpec((1, D), lambda i: (0, 0)),
        ],
        out_specs=pl.BlockSpec((RB, D), lambda i: (i, 0)),
        out_shape=jax.ShapeDtypeStruct((N, D), jnp.float32),
    )(p, p, hs, dinv, b)


def _bn_mm_body(st_ref, t_ref, g_ref, be_ref, w_ref, dv_ref, o_ref):
    s1 = st_ref[0:1, :]
    s2 = st_ref[1:2, :]
    mean = s1 * (1.0 / N)
    var = s2 * (1.0 / N) - mean * mean
    inv = lax.rsqrt(var + BN_EPS)
    a = g_ref[...] * inv
    cc = be_ref[...] - mean * a
    h = jnp.maximum(t_ref[...] * a + cc, 0.0)
    o_ref[...] = jnp.dot(h, w_ref[...], preferred_element_type=jnp.float32) * dv_ref[...]


def _tc_bn_mm(st, t, g, be, W, dinv):
    F, D = W.shape
    return pl.pallas_call(
        _bn_mm_body,
        grid=(GRID,),
        in_specs=[
            pl.BlockSpec((2, F), lambda i: (0, 0)),
            pl.BlockSpec((RB, F), lambda i: (i, 0)),
            pl.BlockSpec((1, F), lambda i: (0, 0)),
            pl.BlockSpec((1, F), lambda i: (0, 0)),
            pl.BlockSpec((F, D), lambda i: (0, 0)),
            pl.BlockSpec((RB, 1), lambda i: (i, 0)),
        ],
        out_specs=pl.BlockSpec((RB, D), lambda i: (i, 0)),
        out_shape=jax.ShapeDtypeStruct((N, D), jnp.float32),
    )(st, t, g, be, W, dinv)


def kernel(x, edge_idx, W0, b0, g0, be0, W1, b1, g1, be1, W2, b2):
    src = edge_idx[0]
    dst = edge_idx[1]
    pad = E_PAD - E
    srcp = jnp.concatenate([src, jnp.zeros((pad,), jnp.int32)]).reshape(NW, CH, K)
    dstp = jnp.concatenate([dst, jnp.full((pad,), NP - 1, jnp.int32)]).reshape(NW, CH, K)
    zero128 = jnp.zeros((K, NHID), jnp.float32)
    ones_blk = jnp.ones((K, DEGW), jnp.float32)
    zero_blk = jnp.zeros((K, DEGW), jnp.float32)

    deg_parts = _sc_deg(dstp, ones_blk, zero_blk)
    dinv = _tc_dinv(deg_parts)

    hs0 = _tc_mm_scale(x, W0, dinv)
    p0 = _sc_prop128(hs0, srcp, dstp, zero128)
    t0, st0 = _tc_combine(p0, hs0, dinv, b0.reshape(1, -1))

    hs1 = _tc_bn_mm(st0, t0, g0.reshape(1, -1), be0.reshape(1, -1), W1, dinv)
    p1 = _sc_prop128(hs1, srcp, dstp, zero128)
    t1, st1 = _tc_combine(p1, hs1, dinv, b1.reshape(1, -1))

    W2p = jnp.pad(W2, ((0, 0), (0, D2P - NCLASS)))
    b2p = jnp.pad(b2, (0, D2P - NCLASS)).reshape(1, -1)
    hs2 = _tc_bn_mm(st1, t1, g1.reshape(1, -1), be1.reshape(1, -1), W2p, dinv)
    p2 = _sc_prop128(hs2, srcp, dstp, zero128)
    out = _tc_combine_final(p2, hs2, dinv, b2p)

    return out[:, :NCLASS]


# speedup vs baseline: 1.1582x; 1.0023x over previous
import functools

import jax
import jax.numpy as jnp
from jax import lax
from jax.experimental import pallas as pl
from jax.experimental.pallas import tpu as pltpu
from jax.experimental.pallas import tpu_sc as plsc

N = 10000
E = 320000
NHID = 128
NCLASS = 40
D2P = 128
BN_EPS = 1e-5

NC = 2
NS = 16
NW = NC * NS
L = 16
NP = 10240
STRIPE = NP // NS
K = 128
CH = 80
CHB = 8
NB = CH // CHB
PER_W = CH * K
E_PAD = NW * PER_W

_SC_MESH = dict(
    mesh=plsc.VectorSubcoreMesh(core_axis_name="c", subcore_axis_name="s"),
)


def _make_prop(D):
    def body(hs_hbm, srcp_hbm, dstp_hbm, zero_hbm, out_hbm,
             sA, dA, sB, dB, b0, b1, acc, g0, g1, iA, iB):
        c = lax.axis_index("c")
        s = lax.axis_index("s")
        w = s * NC + c
        for z in range(STRIPE // K):
            pltpu.sync_copy(zero_hbm, acc.at[pl.ds(s * STRIPE + z * K, K)])
        plsc.subcore_barrier()

        def idx_issue(bi, sv, dv, isem):
            pltpu.async_copy(srcp_hbm.at[w].at[pl.ds(bi * CHB, CHB)], sv, isem)
            pltpu.async_copy(dstp_hbm.at[w].at[pl.ds(bi * CHB, CHB)], dv, isem)

        def idx_wait(bi, sv, dv, isem):
            pltpu.make_async_copy(srcp_hbm.at[w].at[pl.ds(bi * CHB, CHB)],
                                  sv, isem).wait()
            pltpu.make_async_copy(dstp_hbm.at[w].at[pl.ds(bi * CHB, CHB)],
                                  dv, isem).wait()

        pltpu.sync_copy(srcp_hbm.at[w].at[pl.ds(0, CHB)], sA)
        pltpu.sync_copy(dstp_hbm.at[w].at[pl.ds(0, CHB)], dA)
        pltpu.async_copy(hs_hbm.at[sA.at[0]], b0, g0)
        pltpu.async_copy(hs_hbm.at[sA.at[1]], b1, g1)

        def do_block(bi, srcv, dstv, nsv, ndv, nisem):
            @pl.when(bi + 1 < NB)
            def _():
                idx_issue(bi + 1, nsv, ndv, nisem)

            for ci in range(CHB):
                buf, sem = (b0, g0) if ci % 2 == 0 else (b1, g1)
                pltpu.make_async_copy(hs_hbm.at[srcv.at[ci]], buf, sem).wait()
                pltpu.sync_copy(buf, acc.at[dstv.at[ci]], add=True)
                if ci + 2 < CHB:
                    pltpu.async_copy(hs_hbm.at[srcv.at[ci + 2]], buf, sem)
                elif ci == CHB - 2:
                    @pl.when(bi + 1 < NB)
                    def _(buf=buf, sem=sem):
                        idx_wait(bi + 1, nsv, ndv, nisem)
                        pltpu.async_copy(hs_hbm.at[nsv.at[0]], buf, sem)
                else:
                    @pl.when(bi + 1 < NB)
                    def _(buf=buf, sem=sem):
                        pltpu.async_copy(hs_hbm.at[nsv.at[1]], buf, sem)

        def pair_body(pi, carry):
            do_block(pi * 2, sA, dA, sB, dB, iB)
            do_block(pi * 2 + 1, sB, dB, sA, dA, iA)
            return carry

        lax.fori_loop(0, NB // 2, pair_body, 0)
        plsc.subcore_barrier()
        pltpu.sync_copy(acc.at[pl.ds(s * STRIPE, STRIPE)],
                        out_hbm.at[c].at[pl.ds(s * STRIPE, STRIPE)])

    return pl.kernel(
        body,
        out_type=jax.ShapeDtypeStruct((NC, NP, D), jnp.float32),
        scratch_types=[
            pltpu.VMEM((CHB, K), jnp.int32),
            pltpu.VMEM((CHB, K), jnp.int32),
            pltpu.VMEM((CHB, K), jnp.int32),
            pltpu.VMEM((CHB, K), jnp.int32),
            pltpu.VMEM((K, D), jnp.float32),
            pltpu.VMEM((K, D), jnp.float32),
            pltpu.VMEM_SHARED((NP, D), jnp.float32),
            pltpu.SemaphoreType.DMA,
            pltpu.SemaphoreType.DMA,
            pltpu.SemaphoreType.DMA,
            pltpu.SemaphoreType.DMA,
        ],
        **_SC_MESH,
    )


_sc_prop128 = _make_prop(NHID)

DEGW = 128


def _deg_body(dstp_hbm, ones_hbm, zero_hbm, out_hbm, dA, dB, onesv, acc,
              iA, iB):
    c = lax.axis_index("c")
    s = lax.axis_index("s")
    w = s * NC + c
    pltpu.sync_copy(ones_hbm, onesv)
    for z in range(STRIPE // K):
        pltpu.sync_copy(zero_hbm, acc.at[pl.ds(s * STRIPE + z * K, K)])
    plsc.subcore_barrier()

    def idx_issue(bi, dv, isem):
        pltpu.async_copy(dstp_hbm.at[w].at[pl.ds(bi * CHB, CHB)], dv, isem)

    def idx_wait(bi, dv, isem):
        pltpu.make_async_copy(dstp_hbm.at[w].at[pl.ds(bi * CHB, CHB)],
                              dv, isem).wait()

    pltpu.sync_copy(dstp_hbm.at[w].at[pl.ds(0, CHB)], dA)

    def do_block(bi, dstv, nd, nisem):
        @pl.when(bi + 1 < NB)
        def _():
            idx_issue(bi + 1, nd, nisem)

        for ci in range(CHB):
            if ci == CHB - 1:
                @pl.when(bi + 1 < NB)
                def _():
                    idx_wait(bi + 1, nd, nisem)
            pltpu.sync_copy(onesv, acc.at[dstv.at[ci]], add=True)

    def pair_body(pi, carry):
        do_block(pi * 2, dA, dB, iB)
        do_block(pi * 2 + 1, dB, dA, iA)
        return carry

    lax.fori_loop(0, NB // 2, pair_body, 0)
    plsc.subcore_barrier()
    pltpu.sync_copy(acc.at[pl.ds(s * STRIPE, STRIPE)],
                    out_hbm.at[c].at[pl.ds(s * STRIPE, STRIPE)])


_sc_deg = pl.kernel(
    _deg_body,
    out_type=jax.ShapeDtypeStruct((NC, NP, DEGW), jnp.float32),
    scratch_types=[
        pltpu.VMEM((CHB, K), jnp.int32),
        pltpu.VMEM((CHB, K), jnp.int32),
        pltpu.VMEM((K, DEGW), jnp.float32),
        pltpu.VMEM_SHARED((NP, DEGW), jnp.float32),
        pltpu.SemaphoreType.DMA,
        pltpu.SemaphoreType.DMA,
    ],
    **_SC_MESH,
)


RB = 1000
GRID = N // RB


def _dinv_body(dp_ref, o_ref):
    deg = dp_ref[0] + dp_ref[1]
    o_ref[...] = lax.rsqrt(deg[:, 0:1] + 1.0)


def _tc_dinv(deg_parts):
    return pl.pallas_call(
        _dinv_body,
        out_shape=jax.ShapeDtypeStruct((NP, 1), jnp.float32),
    )(deg_parts)


def _mm_body(x_ref, w_ref, dv_ref, o_ref):
    h = jnp.dot(x_ref[...], w_ref[...], preferred_element_type=jnp.float32)
    o_ref[...] = h * dv_ref[...]


def _tc_mm_scale(x, W, dinv):
    F, D = W.shape
    return pl.pallas_call(
        _mm_body,
        grid=(GRID,),
        in_specs=[
            pl.BlockSpec((RB, F), lambda i: (i, 0)),
            pl.BlockSpec((F, D), lambda i: (0, 0)),
            pl.BlockSpec((RB, 1), lambda i: (i, 0)),
        ],
        out_specs=pl.BlockSpec((RB, D), lambda i: (i, 0)),
        out_shape=jax.ShapeDtypeStruct((N, D), jnp.float32),
    )(x, W, dinv)


def _comb_body(p0_ref, p1_ref, hs_ref, dv_ref, b_ref, t_ref, st_ref):
    i = pl.program_id(0)
    t = (p0_ref[0] + p1_ref[0] + hs_ref[...]) * dv_ref[...] + b_ref[...]
    t_ref[...] = t
    s1 = jnp.sum(t, axis=0, keepdims=True)
    s2 = jnp.sum(t * t, axis=0, keepdims=True)
    st = jnp.concatenate([s1, s2], axis=0)

    @pl.when(i == 0)
    def _():
        st_ref[...] = st

    @pl.when(i > 0)
    def _():
        st_ref[...] += st


def _tc_combine(p, hs, dinv, b):
    D = hs.shape[1]
    return pl.pallas_call(
        _comb_body,
        grid=(GRID,),
        in_specs=[
            pl.BlockSpec((1, RB, D), lambda i: (0, i, 0)),
            pl.BlockSpec((1, RB, D), lambda i: (1, i, 0)),
            pl.BlockSpec((RB, D), lambda i: (i, 0)),
            pl.BlockSpec((RB, 1), lambda i: (i, 0)),
            pl.BlockSpec((1, D), lambda i: (0, 0)),
        ],
        out_specs=[
            pl.BlockSpec((RB, D), lambda i: (i, 0)),
            pl.BlockSpec((2, D), lambda i: (0, 0)),
        ],
        out_shape=[
            jax.ShapeDtypeStruct((N, D), jnp.float32),
            jax.ShapeDtypeStruct((2, D), jnp.float32),
        ],
    )(p, p, hs, dinv, b)


def _comb_final_body(p0_ref, p1_ref, hs_ref, dv_ref, b_ref, t_ref):
    t_ref[...] = (p0_ref[0] + p1_ref[0] + hs_ref[...]) * dv_ref[...] + b_ref[...]


def _tc_combine_final(p, hs, dinv, b):
    D = hs.shape[1]
    return pl.pallas_call(
        _comb_final_body,
        grid=(GRID,),
        in_specs=[
            pl.BlockSpec((1, RB, D), lambda i: (0, i, 0)),
            pl.BlockSpec((1, RB, D), lambda i: (1, i, 0)),
            pl.BlockSpec((RB, D), lambda i: (i, 0)),
            pl.BlockSpec((RB, 1), lambda i: (i, 0)),
            pl.BlockSpec((1, D), lambda i: (0, 0)),
        ],
        out_specs=pl.BlockSpec((RB, D), lambda i: (i, 0)),
        out_shape=jax.ShapeDtypeStruct((N, D), jnp.float32),
    )(p, p, hs, dinv, b)


def _bn_mm_body(st_ref, t_ref, g_ref, be_ref, w_ref, dv_ref, o_ref):
    s1 = st_ref[0:1, :]
    s2 = st_ref[1:2, :]
    mean = s1 * (1.0 / N)
    var = s2 * (1.0 / N) - mean * mean
    inv = lax.rsqrt(var + BN_EPS)
    a = g_ref[...] * inv
    cc = be_ref[...] - mean * a
    h = jnp.maximum(t_ref[...] * a + cc, 0.0)
    o_ref[...] = jnp.dot(h, w_ref[...], preferred_element_type=jnp.float32) * dv_ref[...]


def _tc_bn_mm(st, t, g, be, W, dinv):
    F, D = W.shape
    return pl.pallas_call(
        _bn_mm_body,
        grid=(GRID,),
        in_specs=[
            pl.BlockSpec((2, F), lambda i: (0, 0)),
            pl.BlockSpec((RB, F), lambda i: (i, 0)),
            pl.BlockSpec((1, F), lambda i: (0, 0)),
            pl.BlockSpec((1, F), lambda i: (0, 0)),
            pl.BlockSpec((F, D), lambda i: (0, 0)),
            pl.BlockSpec((RB, 1), lambda i: (i, 0)),
        ],
        out_specs=pl.BlockSpec((RB, D), lambda i: (i, 0)),
        out_shape=jax.ShapeDtypeStruct((N, D), jnp.float32),
    )(st, t, g, be, W, dinv)


def kernel(x, edge_idx, W0, b0, g0, be0, W1, b1, g1, be1, W2, b2):
    src = edge_idx[0]
    dst = edge_idx[1]
    pad = E_PAD - E
    srcp = jnp.concatenate([src, jnp.zeros((pad,), jnp.int32)]).reshape(NW, CH, K)
    dstp = jnp.concatenate([dst, jnp.full((pad,), NP - 1, jnp.int32)]).reshape(NW, CH, K)
    zero128 = jnp.zeros((K, NHID), jnp.float32)
    ones_blk = jnp.ones((K, DEGW), jnp.float32)
    zero_blk = jnp.zeros((K, DEGW), jnp.float32)

    deg_parts = _sc_deg(dstp, ones_blk, zero_blk)
    dinv = _tc_dinv(deg_parts)

    hs0 = _tc_mm_scale(x, W0, dinv)
    p0 = _sc_prop128(hs0, srcp, dstp, zero128)
    t0, st0 = _tc_combine(p0, hs0, dinv, b0.reshape(1, -1))

    hs1 = _tc_bn_mm(st0, t0, g0.reshape(1, -1), be0.reshape(1, -1), W1, dinv)
    p1 = _sc_prop128(hs1, srcp, dstp, zero128)
    t1, st1 = _tc_combine(p1, hs1, dinv, b1.reshape(1, -1))

    W2p = jnp.pad(W2, ((0, 0), (0, D2P - NCLASS)))
    b2p = jnp.pad(b2, (0, D2P - NCLASS)).reshape(1, -1)
    hs2 = _tc_bn_mm(st1, t1, g1.reshape(1, -1), be1.reshape(1, -1), W2p, dinv)
    p2 = _sc_prop128(hs2, srcp, dstp, zero128)
    out = _tc_combine_final(p2, hs2, dinv, b2p)

    return out[:, :NCLASS]
